# Initial kernel scaffold; baseline (speedup 1.0000x reference)
import jax
import jax.numpy as jnp
from jax.experimental import pallas as pl


def kernel(x, f, y_i, y_s, n_true, e_true):
    raise NotImplementedError("write your pallas kernel here")


# speedup vs baseline: 2.3674x; 2.3674x over previous
import functools

import jax
import jax.numpy as jnp
from jax import lax
from jax.experimental import pallas as pl
from jax.experimental.pallas import tpu as pltpu
from jax.experimental.pallas import tpu_sc as plsc

N_HIT = 10000
K_OBJ = 256
N_EDGE = 10000
DIM = 8
NPAD = 10240
EPAD = 10240
NW = 32
HPT = NPAD // NW
EPT = EPAD // NW
LANES = 16
BIG_I = 2**30


def _sc_mask_and_gather(eh_pad, ep_pad, f_pad):
    mesh = plsc.VectorSubcoreMesh(core_axis_name="c", subcore_axis_name="s")

    slab_words = HPT * K_OBJ

    @functools.partial(
        pl.kernel,
        mesh=mesh,
        compiler_params=pltpu.CompilerParams(needs_layout_passes=False),
        out_type=[
            jax.ShapeDtypeStruct((NPAD * K_OBJ,), jnp.float32),
            jax.ShapeDtypeStruct((EPAD,), jnp.float32),
        ],
        scratch_types=[
            pltpu.VMEM((slab_words,), jnp.float32),
            pltpu.VMEM((EPAD,), jnp.int32),
            pltpu.VMEM((EPAD,), jnp.int32),
            pltpu.VMEM((NPAD,), jnp.float32),
            pltpu.VMEM((EPT,), jnp.float32),
        ],
    )
    def body(eh_hbm, ep_hbm, f_hbm, mask_hbm, fe_hbm, slab, eh_v, ep_v, f_v, fe_v):
        wid = lax.axis_index("s") * 2 + lax.axis_index("c")
        hbase = wid * HPT
        ebase = wid * EPT

        pltpu.sync_copy(eh_hbm, eh_v)
        pltpu.sync_copy(ep_hbm, ep_v)
        pltpu.sync_copy(f_hbm, f_v)

        zeros16 = jnp.zeros((LANES,), jnp.float32)

        def zero_step(i, carry):
            slab[pl.ds(i * LANES, LANES)] = zeros16
            return carry

        lax.fori_loop(0, slab_words // LANES, zero_step, 0)

        ones16 = jnp.ones((LANES,), jnp.float32)
        lane = jnp.arange(LANES, dtype=jnp.int32)

        def scan_step(i, carry):
            h16 = eh_v[pl.ds(i * LANES, LANES)]
            p16 = ep_v[pl.ds(i * LANES, LANES)]
            evalid = (i * LANES + lane) < N_EDGE
            hl = h16 - hbase
            own = evalid & (hl >= 0) & (hl < HPT)
            hl = jnp.clip(hl, 0, HPT - 1)
            plsc.store_scatter(slab, [hl * K_OBJ + p16], ones16, mask=own)
            return carry

        lax.fori_loop(0, EPAD // LANES, scan_step, 0)

        def gather_step(j, carry):
            idx = eh_v[pl.ds(ebase + j * LANES, LANES)]
            fe_v[pl.ds(j * LANES, LANES)] = plsc.load_gather(f_v, [idx])
            return carry

        lax.fori_loop(0, EPT // LANES, gather_step, 0)

        pltpu.sync_copy(slab, mask_hbm.at[pl.ds(hbase * K_OBJ, slab_words)])
        pltpu.sync_copy(fe_v, fe_hbm.at[pl.ds(ebase, EPT)])

    return body(eh_pad, ep_pad, f_pad)


def _tc_body(x_ref, f_ref, yi_ref, ys_ref, ep_ref, eh_ref, fe_ref, mask_ref,
             out_ref):
    hi = jax.lax.Precision.HIGHEST
    iota_k = lax.broadcasted_iota(jnp.int32, (K_OBJ, 1), 0)
    lane128 = lax.broadcasted_iota(jnp.int32, (1, 128), 1)
    lane1024 = lax.broadcasted_iota(jnp.int32, (1, 1024), 1)

    row10 = lax.broadcasted_iota(jnp.int32, (10, 1024), 0)
    col10 = lax.broadcasted_iota(jnp.int32, (10, 1024), 1)
    hvalid_all = (row10 * 1024 + col10) < N_HIT
    yi = yi_ref[...]
    ys = ys_ref[...]
    fh = f_ref[...]
    bkg = (yi == -1) & (ys >= 0) & hvalid_all
    n_bkg = jnp.sum(jnp.where(bkg, 1.0, 0.0))
    f_bkg = jnp.sum(jnp.where(bkg, fh, 0.0))

    def seg_chunk(c, carry):
        ep_c = ep_ref[pl.ds(c, 1), :]
        fe_c = fe_ref[pl.ds(c, 1), :]
        evalid = (c * 128 + lane128) < N_EDGE
        oh = (ep_c == iota_k) & evalid
        cand = jnp.where(oh, fe_c, -1.0)
        return jnp.maximum(carry, jnp.max(cand, axis=1, keepdims=True))

    seg_max = lax.fori_loop(0, EPAD // 128, seg_chunk,
                            jnp.full((K_OBJ, 1), -1.0, jnp.float32))

    def arg_chunk(c, carry):
        ep_c = ep_ref[pl.ds(c, 1), :]
        fe_c = fe_ref[pl.ds(c, 1), :]
        eidx = c * 128 + lane128
        evalid = eidx < N_EDGE
        ismax = (ep_c == iota_k) & evalid & (fe_c == seg_max)
        cand = jnp.where(ismax, eidx, BIG_I)
        return jnp.minimum(carry, jnp.min(cand, axis=1, keepdims=True))

    cie = lax.fori_loop(0, EPAD // 128, arg_chunk,
                        jnp.full((K_OBJ, 1), BIG_I, jnp.int32))

    def ctr_chunk(c, carry):
        eh_c = eh_ref[pl.ds(c, 1), :]
        eidx = c * 128 + lane128
        sel = eidx == cie
        return carry + jnp.sum(jnp.where(sel, eh_c, 0), axis=1, keepdims=True)

    centers = lax.fori_loop(0, EPAD // 128, ctr_chunk,
                            jnp.zeros((K_OBJ, 1), jnp.int32))

    def xc_chunk(b, carry):
        hidx = b * 1024 + lane1024
        oh = (centers == hidx).astype(jnp.float32)
        xb = x_ref[pl.ds(b * 1024, 1024), :]
        return carry + lax.dot_general(oh, xb, (((1,), (0,)), ((), ())),
                                       precision=hi)

    xc = lax.fori_loop(0, NPAD // 1024, xc_chunk,
                       jnp.zeros((K_OBJ, DIM), jnp.float32))
    xc2 = jnp.sum(xc * xc, axis=1, keepdims=True)

    f_centers = seg_max
    t = 0.5 * jnp.log((1.0 + f_centers) / (1.0 - f_centers))
    qc = t * t + 0.5

    eye_k = (iota_k == lax.broadcasted_iota(jnp.int32, (K_OBJ, K_OBJ), 1))
    eye_k = eye_k.astype(jnp.float32)
    xc2_row = lax.dot_general(xc2, eye_k, (((0,), (0,)), ((), ())),
                              precision=hi)
    qc_row = lax.dot_general(qc, eye_k, (((0,), (0,)), ((), ())),
                             precision=hi)

    def dense_block(b, vacc):
        xb = x_ref[pl.ds(b * 1024, 1024), :]
        x2b = jnp.sum(xb * xb, axis=1, keepdims=True)
        dots = lax.dot_general(xb, xc, (((1,), (1,)), ((), ())), precision=hi)
        dist = x2b + xc2_row - 2.0 * dots
        mb = mask_ref[pl.ds(b * 1024, 1024), :]
        val = jnp.where(mb > 0.5, dist, jnp.maximum(1.0 - dist, 0.0))
        wsum = jnp.sum(val * qc_row, axis=1, keepdims=True)
        f_b = f_ref[pl.ds(b, 1), :]
        hvalid = (b * 1024 + lane1024) < N_HIT
        tq = 0.5 * jnp.log((1.0 + f_b) / (1.0 - f_b))
        q_b = jnp.where(hvalid, tq * tq + 0.5, 0.0)
        contrib = lax.dot_general(q_b, wsum, (((1,), (0,)), ((), ())),
                                  precision=hi)
        return vacc + contrib

    vtot = lax.fori_loop(0, NPAD // 1024, dense_block,
                         jnp.zeros((1, 1), jnp.float32))
    v = vtot[0, 0] / N_HIT

    b_out = (1.0 - jnp.sum(f_centers) / K_OBJ
             + jnp.where(n_bkg > 0.0, f_bkg / jnp.maximum(n_bkg, 1.0), 0.0))

    row8 = lax.broadcasted_iota(jnp.int32, (8, 128), 0)
    col8 = lax.broadcasted_iota(jnp.int32, (8, 128), 1)
    out = jnp.where((row8 == 0) & (col8 == 0), b_out,
                    jnp.where((row8 == 0) & (col8 == 1), v, 0.0))
    out_ref[...] = out


def _tc_call(x, f10, yi10, ys10, ep80, eh80, fe80, mask):
    return pl.pallas_call(
        _tc_body,
        out_shape=jax.ShapeDtypeStruct((8, 128), jnp.float32),
    )(x, f10, yi10, ys10, ep80, eh80, fe80, mask)


def kernel(x, f, y_i, y_s, n_true, e_true):
    eh = e_true[0]
    ep = e_true[1]
    pad_e = EPAD - N_EDGE
    pad_h = NPAD - N_HIT
    eh_pad = jnp.pad(eh, (0, pad_e))
    ep_pad = jnp.pad(ep, (0, pad_e))
    f_pad = jnp.pad(f, (0, pad_h))

    mask_flat, fe = _sc_mask_and_gather(eh_pad, ep_pad, f_pad)
    mask = mask_flat.reshape(NPAD, K_OBJ)

    xp = jnp.pad(x, ((0, pad_h), (0, 0)))
    f10 = f_pad.reshape(10, 1024)
    yi10 = jnp.pad(y_i, (0, pad_h)).reshape(10, 1024)
    ys10 = jnp.pad(y_s, (0, pad_h)).reshape(10, 1024)
    ep80 = ep_pad.reshape(80, 128)
    eh80 = eh_pad.reshape(80, 128)
    fe80 = fe.reshape(80, 128)

    res = _tc_call(xp, f10, yi10, ys10, ep80, eh80, fe80, mask)
    return res[0, 0:2]


# speedup vs baseline: 3.1292x; 1.3218x over previous
import functools

import jax
import jax.numpy as jnp
from jax import lax
from jax.experimental import pallas as pl
from jax.experimental.pallas import tpu as pltpu
from jax.experimental.pallas import tpu_sc as plsc

N_HIT = 10000
K_OBJ = 256
N_EDGE = 10000
DIM = 8
NPAD = 10240
EPAD = 10240
NW = 32
HPT = NPAD // NW
EPT = EPAD // NW
LANES = 16
BIG_I = 2**30


def _sc_mask_and_gather(eh_pad, ep_pad, f_pad):
    mesh = plsc.VectorSubcoreMesh(core_axis_name="c", subcore_axis_name="s")

    half = K_OBJ // 2
    slab_words = HPT * half

    @functools.partial(
        pl.kernel,
        mesh=mesh,
        compiler_params=pltpu.CompilerParams(needs_layout_passes=False),
        out_type=[
            jax.ShapeDtypeStruct((NPAD * half,), jnp.float32),
            jax.ShapeDtypeStruct((NPAD * half,), jnp.float32),
            jax.ShapeDtypeStruct((EPAD,), jnp.float32),
        ],
        scratch_types=[
            pltpu.VMEM((slab_words,), jnp.float32),
            pltpu.VMEM((slab_words,), jnp.float32),
            pltpu.VMEM((EPAD,), jnp.int32),
            pltpu.VMEM((EPAD,), jnp.int32),
            pltpu.VMEM((NPAD,), jnp.float32),
            pltpu.VMEM((EPT,), jnp.float32),
        ],
    )
    def body(eh_hbm, ep_hbm, f_hbm, mlo_hbm, mhi_hbm, fe_hbm,
             slab_lo, slab_hi, eh_v, ep_v, f_v, fe_v):
        wid = lax.axis_index("s") * 2 + lax.axis_index("c")
        hbase = wid * HPT
        ebase = wid * EPT

        pltpu.sync_copy(eh_hbm, eh_v)
        pltpu.sync_copy(ep_hbm, ep_v)
        pltpu.sync_copy(f_hbm, f_v)

        zeros16 = jnp.zeros((LANES,), jnp.float32)

        @functools.partial(plsc.parallel_loop, 0, slab_words // LANES, unroll=8)
        def _(i):
            slab_lo[pl.ds(i * LANES, LANES)] = zeros16
            slab_hi[pl.ds(i * LANES, LANES)] = zeros16

        ones16 = jnp.ones((LANES,), jnp.float32)
        lane = jnp.arange(LANES, dtype=jnp.int32)

        @functools.partial(plsc.parallel_loop, 0, EPAD // LANES, unroll=4)
        def _(i):
            h16 = eh_v[pl.ds(i * LANES, LANES)]
            p16 = ep_v[pl.ds(i * LANES, LANES)]
            evalid = (i * LANES + lane) < N_EDGE
            hl = h16 - hbase
            own = evalid & (hl >= 0) & (hl < HPT)
            hl = jnp.clip(hl, 0, HPT - 1)
            idx16 = hl * half + (p16 & (half - 1))
            plsc.store_scatter(slab_lo, [idx16], ones16, mask=own & (p16 < half))
            plsc.store_scatter(slab_hi, [idx16], ones16, mask=own & (p16 >= half))

        @functools.partial(plsc.parallel_loop, 0, EPT // LANES, unroll=4)
        def _(j):
            idx = eh_v[pl.ds(ebase + j * LANES, LANES)]
            fe_v[pl.ds(j * LANES, LANES)] = plsc.load_gather(f_v, [idx])

        pltpu.sync_copy(slab_lo, mlo_hbm.at[pl.ds(hbase * half, slab_words)])
        pltpu.sync_copy(slab_hi, mhi_hbm.at[pl.ds(hbase * half, slab_words)])
        pltpu.sync_copy(fe_v, fe_hbm.at[pl.ds(ebase, EPT)])

    return body(eh_pad, ep_pad, f_pad)


def _tc_body(x_ref, f_ref, yi_ref, ys_ref, ep_ref, eh_ref, fe_ref,
             mlo_ref, mhi_ref, out_ref):
    hi = jax.lax.Precision.HIGHEST
    iota_k = lax.broadcasted_iota(jnp.int32, (K_OBJ, 1), 0)
    lane128 = lax.broadcasted_iota(jnp.int32, (1, 128), 1)
    lane1024 = lax.broadcasted_iota(jnp.int32, (1, 1024), 1)

    row10 = lax.broadcasted_iota(jnp.int32, (10, 1024), 0)
    col10 = lax.broadcasted_iota(jnp.int32, (10, 1024), 1)
    hvalid_all = (row10 * 1024 + col10) < N_HIT
    yi = yi_ref[...]
    ys = ys_ref[...]
    fh = f_ref[...]
    bkg = (yi == -1) & (ys >= 0) & hvalid_all
    n_bkg = jnp.sum(jnp.where(bkg, 1.0, 0.0))
    f_bkg = jnp.sum(jnp.where(bkg, fh, 0.0))

    def seg_chunk(c, carry):
        ep_c = ep_ref[pl.ds(c, 1), :]
        fe_c = fe_ref[pl.ds(c, 1), :]
        evalid = (c * 128 + lane128) < N_EDGE
        oh = (ep_c == iota_k) & evalid
        cand = jnp.where(oh, fe_c, -1.0)
        return jnp.maximum(carry, jnp.max(cand, axis=1, keepdims=True))

    seg_max = lax.fori_loop(0, EPAD // 128, seg_chunk,
                            jnp.full((K_OBJ, 1), -1.0, jnp.float32))

    def arg_chunk(c, carry):
        ep_c = ep_ref[pl.ds(c, 1), :]
        fe_c = fe_ref[pl.ds(c, 1), :]
        eidx = c * 128 + lane128
        evalid = eidx < N_EDGE
        ismax = (ep_c == iota_k) & evalid & (fe_c == seg_max)
        cand = jnp.where(ismax, eidx, BIG_I)
        return jnp.minimum(carry, jnp.min(cand, axis=1, keepdims=True))

    cie = lax.fori_loop(0, EPAD // 128, arg_chunk,
                        jnp.full((K_OBJ, 1), BIG_I, jnp.int32))

    def ctr_chunk(c, carry):
        eh_c = eh_ref[pl.ds(c, 1), :]
        eidx = c * 128 + lane128
        sel = eidx == cie
        return carry + jnp.sum(jnp.where(sel, eh_c, 0), axis=1, keepdims=True)

    centers = lax.fori_loop(0, EPAD // 128, ctr_chunk,
                            jnp.zeros((K_OBJ, 1), jnp.int32))

    def xc_chunk(b, carry):
        hidx = b * 1024 + lane1024
        oh = (centers == hidx).astype(jnp.float32)
        xb = x_ref[pl.ds(b * 1024, 1024), :]
        return carry + lax.dot_general(oh, xb, (((1,), (0,)), ((), ())),
                                       precision=hi)

    xc = lax.fori_loop(0, NPAD // 1024, xc_chunk,
                       jnp.zeros((K_OBJ, DIM), jnp.float32))
    xc2 = jnp.sum(xc * xc, axis=1, keepdims=True)

    f_centers = seg_max
    t = 0.5 * jnp.log((1.0 + f_centers) / (1.0 - f_centers))
    qc = t * t + 0.5

    eye_k = (iota_k == lax.broadcasted_iota(jnp.int32, (K_OBJ, K_OBJ), 1))
    eye_k = eye_k.astype(jnp.float32)
    xc2_row = lax.dot_general(xc2, eye_k, (((0,), (0,)), ((), ())),
                              precision=hi)
    qc_row = lax.dot_general(qc, eye_k, (((0,), (0,)), ((), ())),
                             precision=hi)

    qc_lo = lax.slice(qc_row, (0, 0), (1, 128))
    qc_hi = lax.slice(qc_row, (0, 128), (1, 256))

    def dense_block(b, vacc):
        xb = x_ref[pl.ds(b * 1024, 1024), :]
        x2b = jnp.sum(xb * xb, axis=1, keepdims=True)
        dots = lax.dot_general(xb, xc, (((1,), (1,)), ((), ())), precision=hi)
        dist = x2b + xc2_row - 2.0 * dots
        d_lo = lax.slice(dist, (0, 0), (1024, 128))
        d_hi = lax.slice(dist, (0, 128), (1024, 256))
        mlo = mlo_ref[pl.ds(b * 1024, 1024), :]
        mhi = mhi_ref[pl.ds(b * 1024, 1024), :]
        v_lo = jnp.where(mlo > 0.5, d_lo, jnp.maximum(1.0 - d_lo, 0.0))
        v_hi = jnp.where(mhi > 0.5, d_hi, jnp.maximum(1.0 - d_hi, 0.0))
        wsum = (jnp.sum(v_lo * qc_lo, axis=1, keepdims=True)
                + jnp.sum(v_hi * qc_hi, axis=1, keepdims=True))
        f_b = f_ref[pl.ds(b, 1), :]
        hvalid = (b * 1024 + lane1024) < N_HIT
        tq = 0.5 * jnp.log((1.0 + f_b) / (1.0 - f_b))
        q_b = jnp.where(hvalid, tq * tq + 0.5, 0.0)
        contrib = lax.dot_general(q_b, wsum, (((1,), (0,)), ((), ())),
                                  precision=hi)
        return vacc + contrib

    vtot = lax.fori_loop(0, NPAD // 1024, dense_block,
                         jnp.zeros((1, 1), jnp.float32))
    v = vtot[0, 0] / N_HIT

    b_out = (1.0 - jnp.sum(f_centers) / K_OBJ
             + jnp.where(n_bkg > 0.0, f_bkg / jnp.maximum(n_bkg, 1.0), 0.0))

    row8 = lax.broadcasted_iota(jnp.int32, (8, 128), 0)
    col8 = lax.broadcasted_iota(jnp.int32, (8, 128), 1)
    out = jnp.where((row8 == 0) & (col8 == 0), b_out,
                    jnp.where((row8 == 0) & (col8 == 1), v, 0.0))
    out_ref[...] = out


def _tc_call(x, f10, yi10, ys10, ep80, eh80, fe80, mlo, mhi):
    return pl.pallas_call(
        _tc_body,
        out_shape=jax.ShapeDtypeStruct((8, 128), jnp.float32),
    )(x, f10, yi10, ys10, ep80, eh80, fe80, mlo, mhi)


def kernel(x, f, y_i, y_s, n_true, e_true):
    eh = e_true[0]
    ep = e_true[1]
    pad_e = EPAD - N_EDGE
    pad_h = NPAD - N_HIT
    eh_pad = jnp.pad(eh, (0, pad_e))
    ep_pad = jnp.pad(ep, (0, pad_e))
    f_pad = jnp.pad(f, (0, pad_h))

    mlo_flat, mhi_flat, fe = _sc_mask_and_gather(eh_pad, ep_pad, f_pad)
    mlo = mlo_flat.reshape(NPAD, 128)
    mhi = mhi_flat.reshape(NPAD, 128)

    xp = jnp.pad(x, ((0, pad_h), (0, 0)))
    f10 = f_pad.reshape(10, 1024)
    yi10 = jnp.pad(y_i, (0, pad_h)).reshape(10, 1024)
    ys10 = jnp.pad(y_s, (0, pad_h)).reshape(10, 1024)
    ep80 = ep_pad.reshape(80, 128)
    eh80 = eh_pad.reshape(80, 128)
    fe80 = fe.reshape(80, 128)

    res = _tc_call(xp, f10, yi10, ys10, ep80, eh80, fe80, mlo, mhi)
    return res[0, 0:2]


# speedup vs baseline: 4.8140x; 1.5384x over previous
import functools

import jax
import jax.numpy as jnp
from jax import lax
from jax.experimental import pallas as pl
from jax.experimental.pallas import tpu as pltpu
from jax.experimental.pallas import tpu_sc as plsc

N_HIT = 10000
K_OBJ = 256
N_EDGE = 10000
DIM = 8
NPAD = 10240
EPAD = 10240
NW = 32
HPT = NPAD // NW
EPT = EPAD // NW
LANES = 16
BIG_I = 2**30


def _sc_mask_and_gather(eh_pad, ep_pad, f_pad):
    mesh = plsc.VectorSubcoreMesh(core_axis_name="c", subcore_axis_name="s")

    half = K_OBJ // 2
    slab_words = HPT * half

    @functools.partial(
        pl.kernel,
        mesh=mesh,
        compiler_params=pltpu.CompilerParams(needs_layout_passes=False),
        out_type=[
            jax.ShapeDtypeStruct((NPAD * half,), jnp.float32),
            jax.ShapeDtypeStruct((NPAD * half,), jnp.float32),
            jax.ShapeDtypeStruct((EPAD,), jnp.float32),
        ],
        scratch_types=[
            pltpu.VMEM((slab_words,), jnp.float32),
            pltpu.VMEM((slab_words,), jnp.float32),
            pltpu.VMEM((EPAD,), jnp.int32),
            pltpu.VMEM((EPAD,), jnp.int32),
            pltpu.VMEM((NPAD,), jnp.float32),
            pltpu.VMEM((EPT,), jnp.float32),
        ],
    )
    def body(eh_hbm, ep_hbm, f_hbm, mlo_hbm, mhi_hbm, fe_hbm,
             slab_lo, slab_hi, eh_v, ep_v, f_v, fe_v):
        wid = lax.axis_index("s") * 2 + lax.axis_index("c")
        hbase = wid * HPT
        ebase = wid * EPT

        pltpu.sync_copy(eh_hbm, eh_v)
        pltpu.sync_copy(ep_hbm, ep_v)
        pltpu.sync_copy(f_hbm, f_v)

        zeros16 = jnp.zeros((LANES,), jnp.float32)

        @functools.partial(plsc.parallel_loop, 0, slab_words // LANES, unroll=8)
        def _(i):
            slab_lo[pl.ds(i * LANES, LANES)] = zeros16
            slab_hi[pl.ds(i * LANES, LANES)] = zeros16

        ones16 = jnp.ones((LANES,), jnp.float32)
        lane = jnp.arange(LANES, dtype=jnp.int32)

        @functools.partial(plsc.parallel_loop, 0, EPAD // LANES, unroll=4)
        def _(i):
            h16 = eh_v[pl.ds(i * LANES, LANES)]
            p16 = ep_v[pl.ds(i * LANES, LANES)]
            evalid = (i * LANES + lane) < N_EDGE
            hl = h16 - hbase
            own = evalid & (hl >= 0) & (hl < HPT)
            hl = jnp.clip(hl, 0, HPT - 1)
            idx16 = hl * half + (p16 & (half - 1))
            plsc.store_scatter(slab_lo, [idx16], ones16, mask=own & (p16 < half))
            plsc.store_scatter(slab_hi, [idx16], ones16, mask=own & (p16 >= half))

        @functools.partial(plsc.parallel_loop, 0, EPT // LANES, unroll=4)
        def _(j):
            idx = eh_v[pl.ds(ebase + j * LANES, LANES)]
            fe_v[pl.ds(j * LANES, LANES)] = plsc.load_gather(f_v, [idx])

        pltpu.sync_copy(slab_lo, mlo_hbm.at[pl.ds(hbase * half, slab_words)])
        pltpu.sync_copy(slab_hi, mhi_hbm.at[pl.ds(hbase * half, slab_words)])
        pltpu.sync_copy(fe_v, fe_hbm.at[pl.ds(ebase, EPT)])

    return body(eh_pad, ep_pad, f_pad)


def _tc_body(x_ref, f_ref, yi_ref, ys_ref, ep_ref, eh_ref, fe_ref,
             mlo_ref, mhi_ref, out_ref):
    hi = jax.lax.Precision.HIGHEST
    iota_k = lax.broadcasted_iota(jnp.int32, (K_OBJ, 1), 0)
    lane128 = lax.broadcasted_iota(jnp.int32, (1, 128), 1)
    lane1024 = lax.broadcasted_iota(jnp.int32, (1, 1024), 1)

    row10 = lax.broadcasted_iota(jnp.int32, (10, 1024), 0)
    col10 = lax.broadcasted_iota(jnp.int32, (10, 1024), 1)
    hvalid_all = (row10 * 1024 + col10) < N_HIT
    yi = yi_ref[...]
    ys = ys_ref[...]
    fh = f_ref[...]
    bkg = (yi == -1) & (ys >= 0) & hvalid_all
    n_bkg = jnp.sum(jnp.where(bkg, 1.0, 0.0))
    f_bkg = jnp.sum(jnp.where(bkg, fh, 0.0))

    def seg_chunk(c, carry):
        ep_c = ep_ref[pl.ds(c, 1), :]
        fe_c = fe_ref[pl.ds(c, 1), :]
        evalid = (c * 128 + lane128) < N_EDGE
        oh = (ep_c == iota_k) & evalid
        return jnp.maximum(carry, jnp.where(oh, fe_c, -1.0))

    seg_elem = lax.fori_loop(0, EPAD // 128, seg_chunk,
                             jnp.full((K_OBJ, 128), -1.0, jnp.float32))
    seg_max = jnp.max(seg_elem, axis=1, keepdims=True)

    def arg_chunk(c, carry):
        ep_c = ep_ref[pl.ds(c, 1), :]
        fe_c = fe_ref[pl.ds(c, 1), :]
        eidx = c * 128 + lane128
        evalid = eidx < N_EDGE
        ismax = (ep_c == iota_k) & evalid & (fe_c == seg_max)
        return jnp.minimum(carry, jnp.where(ismax, eidx, BIG_I))

    cie_elem = lax.fori_loop(0, EPAD // 128, arg_chunk,
                             jnp.full((K_OBJ, 128), BIG_I, jnp.int32))
    cie = jnp.min(cie_elem, axis=1, keepdims=True)

    def ctr_chunk(c, carry):
        eh_c = eh_ref[pl.ds(c, 1), :]
        eidx = c * 128 + lane128
        sel = eidx == cie
        return carry + jnp.where(sel, eh_c, 0)

    ctr_elem = lax.fori_loop(0, EPAD // 128, ctr_chunk,
                             jnp.zeros((K_OBJ, 128), jnp.int32))
    centers = jnp.sum(ctr_elem, axis=1, keepdims=True)

    def xc_chunk(b, carry):
        hidx = b * 1024 + lane1024
        oh16 = (centers == hidx).astype(jnp.bfloat16)
        xb = x_ref[pl.ds(b * 1024, 1024), :]
        xb_hi = xb.astype(jnp.bfloat16)
        xb_lo = (xb - xb_hi.astype(jnp.float32)).astype(jnp.bfloat16)
        dn = (((1,), (0,)), ((), ()))
        acc = lax.dot_general(oh16, xb_hi, dn,
                              preferred_element_type=jnp.float32)
        acc = acc + lax.dot_general(oh16, xb_lo, dn,
                                    preferred_element_type=jnp.float32)
        return carry + acc

    xc = lax.fori_loop(0, NPAD // 1024, xc_chunk,
                       jnp.zeros((K_OBJ, DIM), jnp.float32))
    xc2 = jnp.sum(xc * xc, axis=1, keepdims=True)
    xc_hi16 = xc.astype(jnp.bfloat16)
    xc_lo16 = (xc - xc_hi16.astype(jnp.float32)).astype(jnp.bfloat16)

    f_centers = seg_max
    t = 0.5 * jnp.log((1.0 + f_centers) / (1.0 - f_centers))
    qc = t * t + 0.5

    eye_k = (iota_k == lax.broadcasted_iota(jnp.int32, (K_OBJ, K_OBJ), 1))
    eye_k = eye_k.astype(jnp.float32)
    xc2_row = lax.dot_general(xc2, eye_k, (((0,), (0,)), ((), ())),
                              precision=hi)
    qc_row = lax.dot_general(qc, eye_k, (((0,), (0,)), ((), ())),
                             precision=hi)

    qc_lo = lax.slice(qc_row, (0, 0), (1, 128))
    qc_hi = lax.slice(qc_row, (0, 128), (1, 256))

    def dense_block(b, vacc):
        xb = x_ref[pl.ds(b * 1024, 1024), :]
        x2b = jnp.sum(xb * xb, axis=1, keepdims=True)
        xb_hi = xb.astype(jnp.bfloat16)
        xb_lo = (xb - xb_hi.astype(jnp.float32)).astype(jnp.bfloat16)
        dn = (((1,), (1,)), ((), ()))
        dots = (lax.dot_general(xb_hi, xc_hi16, dn,
                                preferred_element_type=jnp.float32)
                + lax.dot_general(xb_hi, xc_lo16, dn,
                                  preferred_element_type=jnp.float32)
                + lax.dot_general(xb_lo, xc_hi16, dn,
                                  preferred_element_type=jnp.float32))
        dist = x2b + xc2_row - 2.0 * dots
        d_lo = lax.slice(dist, (0, 0), (1024, 128))
        d_hi = lax.slice(dist, (0, 128), (1024, 256))
        mlo = mlo_ref[pl.ds(b * 1024, 1024), :]
        mhi = mhi_ref[pl.ds(b * 1024, 1024), :]
        v_lo = jnp.where(mlo > 0.5, d_lo, jnp.maximum(1.0 - d_lo, 0.0))
        v_hi = jnp.where(mhi > 0.5, d_hi, jnp.maximum(1.0 - d_hi, 0.0))
        wsum = (jnp.sum(v_lo * qc_lo, axis=1, keepdims=True)
                + jnp.sum(v_hi * qc_hi, axis=1, keepdims=True))
        f_b = f_ref[pl.ds(b, 1), :]
        hvalid = (b * 1024 + lane1024) < N_HIT
        tq = 0.5 * jnp.log((1.0 + f_b) / (1.0 - f_b))
        q_b = jnp.where(hvalid, tq * tq + 0.5, 0.0)
        contrib = lax.dot_general(q_b, wsum, (((1,), (0,)), ((), ())),
                                  precision=hi)
        return vacc + contrib

    vtot = lax.fori_loop(0, NPAD // 1024, dense_block,
                         jnp.zeros((1, 1), jnp.float32))
    v = vtot[0, 0] / N_HIT

    b_out = (1.0 - jnp.sum(f_centers) / K_OBJ
             + jnp.where(n_bkg > 0.0, f_bkg / jnp.maximum(n_bkg, 1.0), 0.0))

    row8 = lax.broadcasted_iota(jnp.int32, (8, 128), 0)
    col8 = lax.broadcasted_iota(jnp.int32, (8, 128), 1)
    out = jnp.where((row8 == 0) & (col8 == 0), b_out,
                    jnp.where((row8 == 0) & (col8 == 1), v, 0.0))
    out_ref[...] = out


def _tc_call(x, f10, yi10, ys10, ep80, eh80, fe80, mlo, mhi):
    return pl.pallas_call(
        _tc_body,
        out_shape=jax.ShapeDtypeStruct((8, 128), jnp.float32),
    )(x, f10, yi10, ys10, ep80, eh80, fe80, mlo, mhi)


def kernel(x, f, y_i, y_s, n_true, e_true):
    eh = e_true[0]
    ep = e_true[1]
    pad_e = EPAD - N_EDGE
    pad_h = NPAD - N_HIT
    eh_pad = jnp.pad(eh, (0, pad_e))
    ep_pad = jnp.pad(ep, (0, pad_e))
    f_pad = jnp.pad(f, (0, pad_h))

    mlo_flat, mhi_flat, fe = _sc_mask_and_gather(eh_pad, ep_pad, f_pad)
    mlo = mlo_flat.reshape(NPAD, 128)
    mhi = mhi_flat.reshape(NPAD, 128)

    xp = jnp.pad(x, ((0, pad_h), (0, 0)))
    f10 = f_pad.reshape(10, 1024)
    yi10 = jnp.pad(y_i, (0, pad_h)).reshape(10, 1024)
    ys10 = jnp.pad(y_s, (0, pad_h)).reshape(10, 1024)
    ep80 = ep_pad.reshape(80, 128)
    eh80 = eh_pad.reshape(80, 128)
    fe80 = fe.reshape(80, 128)

    res = _tc_call(xp, f10, yi10, ys10, ep80, eh80, fe80, mlo, mhi)
    return res[0, 0:2]


# speedup vs baseline: 4.9949x; 1.0376x over previous
import functools

import jax
import jax.numpy as jnp
from jax import lax
from jax.experimental import pallas as pl
from jax.experimental.pallas import tpu as pltpu
from jax.experimental.pallas import tpu_sc as plsc

N_HIT = 10000
K_OBJ = 256
N_EDGE = 10000
DIM = 8
NPAD = 10240
EPAD = 10240
NW = 32
HPT = NPAD // NW
EPT = EPAD // NW
LANES = 16
BIG_I = 2**30


def _sc_mask_and_gather(eh_pad, ep_pad, f_pad):
    mesh = plsc.VectorSubcoreMesh(core_axis_name="c", subcore_axis_name="s")

    half = K_OBJ // 2
    slab_words = HPT * half

    @functools.partial(
        pl.kernel,
        mesh=mesh,
        compiler_params=pltpu.CompilerParams(needs_layout_passes=False),
        out_type=[
            jax.ShapeDtypeStruct((NPAD * half,), jnp.float32),
            jax.ShapeDtypeStruct((NPAD * half,), jnp.float32),
            jax.ShapeDtypeStruct((EPAD,), jnp.float32),
        ],
        scratch_types=[
            pltpu.VMEM((slab_words,), jnp.float32),
            pltpu.VMEM((slab_words,), jnp.float32),
            pltpu.VMEM((EPAD,), jnp.int32),
            pltpu.VMEM((EPAD,), jnp.int32),
            pltpu.VMEM((NPAD,), jnp.float32),
            pltpu.VMEM((EPT,), jnp.float32),
        ],
    )
    def body(eh_hbm, ep_hbm, f_hbm, mlo_hbm, mhi_hbm, fe_hbm,
             slab_lo, slab_hi, eh_v, ep_v, f_v, fe_v):
        wid = lax.axis_index("s") * 2 + lax.axis_index("c")
        hbase = wid * HPT
        ebase = wid * EPT

        pltpu.sync_copy(eh_hbm, eh_v)
        pltpu.sync_copy(ep_hbm, ep_v)
        pltpu.sync_copy(f_hbm, f_v)

        zeros16 = jnp.zeros((LANES,), jnp.float32)

        @functools.partial(plsc.parallel_loop, 0, slab_words // LANES, unroll=16)
        def _(i):
            slab_lo[pl.ds(i * LANES, LANES)] = zeros16
            slab_hi[pl.ds(i * LANES, LANES)] = zeros16

        ones16 = jnp.ones((LANES,), jnp.float32)

        @functools.partial(plsc.parallel_loop, 0, N_EDGE // LANES, unroll=5)
        def _(i):
            h16 = eh_v[pl.ds(i * LANES, LANES)]
            p16 = ep_v[pl.ds(i * LANES, LANES)]
            hl = h16 - hbase
            own = (hl >= 0) & (hl < HPT)
            hl = jnp.clip(hl, 0, HPT - 1)
            idx16 = hl * half + (p16 & (half - 1))
            plsc.store_scatter(slab_lo, [idx16], ones16, mask=own & (p16 < half))
            plsc.store_scatter(slab_hi, [idx16], ones16, mask=own & (p16 >= half))

        @functools.partial(plsc.parallel_loop, 0, EPT // LANES, unroll=4)
        def _(j):
            idx = eh_v[pl.ds(ebase + j * LANES, LANES)]
            fe_v[pl.ds(j * LANES, LANES)] = plsc.load_gather(f_v, [idx])

        pltpu.sync_copy(slab_lo, mlo_hbm.at[pl.ds(hbase * half, slab_words)])
        pltpu.sync_copy(slab_hi, mhi_hbm.at[pl.ds(hbase * half, slab_words)])
        pltpu.sync_copy(fe_v, fe_hbm.at[pl.ds(ebase, EPT)])

    return body(eh_pad, ep_pad, f_pad)


def _tc_body(x_ref, f_ref, yi_ref, ys_ref, ep_ref, eh_ref, fe_ref,
             mlo_ref, mhi_ref, out_ref):
    hi = jax.lax.Precision.HIGHEST
    iota_k = lax.broadcasted_iota(jnp.int32, (K_OBJ, 1), 0)
    lane128 = lax.broadcasted_iota(jnp.int32, (1, 128), 1)
    lane1024 = lax.broadcasted_iota(jnp.int32, (1, 1024), 1)

    row10 = lax.broadcasted_iota(jnp.int32, (10, 1024), 0)
    col10 = lax.broadcasted_iota(jnp.int32, (10, 1024), 1)
    hvalid_all = (row10 * 1024 + col10) < N_HIT
    yi = yi_ref[...]
    ys = ys_ref[...]
    fh = f_ref[...]
    bkg = (yi == -1) & (ys >= 0) & hvalid_all
    n_bkg = jnp.sum(jnp.where(bkg, 1.0, 0.0))
    f_bkg = jnp.sum(jnp.where(bkg, fh, 0.0))

    def seg_chunk(c, carry):
        M, MI, EH = carry
        ep_c = ep_ref[pl.ds(c, 1), :]
        fe_c = fe_ref[pl.ds(c, 1), :]
        eh_c = eh_ref[pl.ds(c, 1), :]
        evalid = (c * 128 + lane128) < N_EDGE
        oh = (ep_c == iota_k) & evalid
        cand = jnp.where(oh, fe_c, -1.0)
        better = cand > M
        eidx = c * 128 + lane128
        return (jnp.where(better, cand, M),
                jnp.where(better, eidx, MI),
                jnp.where(better, eh_c, EH))

    M, MI, EH = lax.fori_loop(
        0, EPAD // 128, seg_chunk,
        (jnp.full((K_OBJ, 128), -1.0, jnp.float32),
         jnp.full((K_OBJ, 128), BIG_I, jnp.int32),
         jnp.zeros((K_OBJ, 128), jnp.int32)))
    seg_max = jnp.max(M, axis=1, keepdims=True)
    mi_m = jnp.where(M == seg_max, MI, BIG_I)
    gmi = jnp.min(mi_m, axis=1, keepdims=True)
    sel = mi_m == gmi
    centers = jnp.sum(jnp.where(sel, EH, 0), axis=1, keepdims=True)

    def xc_chunk(b, carry):
        hidx = b * 1024 + lane1024
        oh16 = (centers == hidx).astype(jnp.bfloat16)
        xb = x_ref[pl.ds(b * 1024, 1024), :]
        xb_hi = xb.astype(jnp.bfloat16)
        xb_lo = (xb - xb_hi.astype(jnp.float32)).astype(jnp.bfloat16)
        dn = (((1,), (0,)), ((), ()))
        acc = lax.dot_general(oh16, xb_hi, dn,
                              preferred_element_type=jnp.float32)
        acc = acc + lax.dot_general(oh16, xb_lo, dn,
                                    preferred_element_type=jnp.float32)
        return carry + acc

    xc = lax.fori_loop(0, NPAD // 1024, xc_chunk,
                       jnp.zeros((K_OBJ, DIM), jnp.float32))
    xc2 = jnp.sum(xc * xc, axis=1, keepdims=True)
    xc_hi16 = xc.astype(jnp.bfloat16)
    xc_lo16 = (xc - xc_hi16.astype(jnp.float32)).astype(jnp.bfloat16)

    f_centers = seg_max
    t = 0.5 * jnp.log((1.0 + f_centers) / (1.0 - f_centers))
    qc = t * t + 0.5

    eye_k = (iota_k == lax.broadcasted_iota(jnp.int32, (K_OBJ, K_OBJ), 1))
    eye_k = eye_k.astype(jnp.float32)
    xc2_row = lax.dot_general(xc2, eye_k, (((0,), (0,)), ((), ())),
                              precision=hi)
    qc_row = lax.dot_general(qc, eye_k, (((0,), (0,)), ((), ())),
                             precision=hi)

    qc_lo = lax.slice(qc_row, (0, 0), (1, 128))
    qc_hi = lax.slice(qc_row, (0, 128), (1, 256))

    def dense_block(b, vacc):
        xb = x_ref[pl.ds(b * 1024, 1024), :]
        x2b = jnp.sum(xb * xb, axis=1, keepdims=True)
        xb_hi = xb.astype(jnp.bfloat16)
        xb_lo = (xb - xb_hi.astype(jnp.float32)).astype(jnp.bfloat16)
        dn = (((1,), (1,)), ((), ()))
        dots = (lax.dot_general(xb_hi, xc_hi16, dn,
                                preferred_element_type=jnp.float32)
                + lax.dot_general(xb_hi, xc_lo16, dn,
                                  preferred_element_type=jnp.float32)
                + lax.dot_general(xb_lo, xc_hi16, dn,
                                  preferred_element_type=jnp.float32))
        dist = x2b + xc2_row - 2.0 * dots
        d_lo = lax.slice(dist, (0, 0), (1024, 128))
        d_hi = lax.slice(dist, (0, 128), (1024, 256))
        mlo = mlo_ref[pl.ds(b * 1024, 1024), :]
        mhi = mhi_ref[pl.ds(b * 1024, 1024), :]
        v_lo = jnp.where(mlo > 0.5, d_lo, jnp.maximum(1.0 - d_lo, 0.0))
        v_hi = jnp.where(mhi > 0.5, d_hi, jnp.maximum(1.0 - d_hi, 0.0))
        wsum = (jnp.sum(v_lo * qc_lo, axis=1, keepdims=True)
                + jnp.sum(v_hi * qc_hi, axis=1, keepdims=True))
        f_b = f_ref[pl.ds(b, 1), :]
        hvalid = (b * 1024 + lane1024) < N_HIT
        tq = 0.5 * jnp.log((1.0 + f_b) / (1.0 - f_b))
        q_b = jnp.where(hvalid, tq * tq + 0.5, 0.0)
        contrib = lax.dot_general(q_b, wsum, (((1,), (0,)), ((), ())),
                                  precision=hi)
        return vacc + contrib

    vtot = lax.fori_loop(0, NPAD // 1024, dense_block,
                         jnp.zeros((1, 1), jnp.float32))
    v = vtot[0, 0] / N_HIT

    b_out = (1.0 - jnp.sum(f_centers) / K_OBJ
             + jnp.where(n_bkg > 0.0, f_bkg / jnp.maximum(n_bkg, 1.0), 0.0))

    row8 = lax.broadcasted_iota(jnp.int32, (8, 128), 0)
    col8 = lax.broadcasted_iota(jnp.int32, (8, 128), 1)
    out = jnp.where((row8 == 0) & (col8 == 0), b_out,
                    jnp.where((row8 == 0) & (col8 == 1), v, 0.0))
    out_ref[...] = out


def _tc_call(x, f10, yi10, ys10, ep80, eh80, fe80, mlo, mhi):
    return pl.pallas_call(
        _tc_body,
        out_shape=jax.ShapeDtypeStruct((8, 128), jnp.float32),
    )(x, f10, yi10, ys10, ep80, eh80, fe80, mlo, mhi)


def kernel(x, f, y_i, y_s, n_true, e_true):
    eh = e_true[0]
    ep = e_true[1]
    pad_e = EPAD - N_EDGE
    pad_h = NPAD - N_HIT
    eh_pad = jnp.pad(eh, (0, pad_e))
    ep_pad = jnp.pad(ep, (0, pad_e))
    f_pad = jnp.pad(f, (0, pad_h))

    mlo_flat, mhi_flat, fe = _sc_mask_and_gather(eh_pad, ep_pad, f_pad)
    mlo = mlo_flat.reshape(NPAD, 128)
    mhi = mhi_flat.reshape(NPAD, 128)

    xp = jnp.pad(x, ((0, pad_h), (0, 0)))
    f10 = f_pad.reshape(10, 1024)
    yi10 = jnp.pad(y_i, (0, pad_h)).reshape(10, 1024)
    ys10 = jnp.pad(y_s, (0, pad_h)).reshape(10, 1024)
    ep80 = ep_pad.reshape(80, 128)
    eh80 = eh_pad.reshape(80, 128)
    fe80 = fe.reshape(80, 128)

    res = _tc_call(xp, f10, yi10, ys10, ep80, eh80, fe80, mlo, mhi)
    return res[0, 0:2]


# speedup vs baseline: 5.3887x; 1.0788x over previous
import functools

import jax
import jax.numpy as jnp
from jax import lax
from jax.experimental import pallas as pl
from jax.experimental.pallas import tpu as pltpu
from jax.experimental.pallas import tpu_sc as plsc

N_HIT = 10000
K_OBJ = 256
N_EDGE = 10000
DIM = 8
NPAD = 10240
EPAD = 10240
NW = 32
HPT = NPAD // NW
EPT = EPAD // NW
LANES = 16
BIG_I = 2**30


def _sc_mask_and_gather(eh_pad, ep_pad, f_pad):
    mesh = plsc.VectorSubcoreMesh(core_axis_name="c", subcore_axis_name="s")

    half = K_OBJ // 2
    slab_words = HPT * half

    @functools.partial(
        pl.kernel,
        mesh=mesh,
        compiler_params=pltpu.CompilerParams(needs_layout_passes=False),
        out_type=[
            jax.ShapeDtypeStruct((NPAD * half,), jnp.float32),
            jax.ShapeDtypeStruct((NPAD * half,), jnp.float32),
            jax.ShapeDtypeStruct((EPAD,), jnp.float32),
        ],
        scratch_types=[
            pltpu.VMEM((slab_words,), jnp.float32),
            pltpu.VMEM((slab_words,), jnp.float32),
            pltpu.VMEM((EPAD,), jnp.int32),
            pltpu.VMEM((EPAD,), jnp.int32),
            pltpu.VMEM((NPAD,), jnp.float32),
            pltpu.VMEM((EPT,), jnp.float32),
        ],
    )
    def body(eh_hbm, ep_hbm, f_hbm, mlo_hbm, mhi_hbm, fe_hbm,
             slab_lo, slab_hi, eh_v, ep_v, f_v, fe_v):
        wid = lax.axis_index("s") * 2 + lax.axis_index("c")
        hbase = wid * HPT
        ebase = wid * EPT

        pltpu.sync_copy(eh_hbm, eh_v)
        pltpu.sync_copy(ep_hbm, ep_v)
        pltpu.sync_copy(f_hbm, f_v)

        zeros16 = jnp.zeros((LANES,), jnp.float32)

        @functools.partial(plsc.parallel_loop, 0, slab_words // LANES, unroll=16)
        def _(i):
            slab_lo[pl.ds(i * LANES, LANES)] = zeros16
            slab_hi[pl.ds(i * LANES, LANES)] = zeros16

        ones16 = jnp.ones((LANES,), jnp.float32)

        @functools.partial(plsc.parallel_loop, 0, N_EDGE // LANES, unroll=5)
        def _(i):
            h16 = eh_v[pl.ds(i * LANES, LANES)]
            p16 = ep_v[pl.ds(i * LANES, LANES)]
            hl = h16 - hbase
            own = (hl >= 0) & (hl < HPT)
            hl = jnp.clip(hl, 0, HPT - 1)
            idx16 = hl * half + (p16 & (half - 1))
            plsc.store_scatter(slab_lo, [idx16], ones16, mask=own & (p16 < half))
            plsc.store_scatter(slab_hi, [idx16], ones16, mask=own & (p16 >= half))

        @functools.partial(plsc.parallel_loop, 0, EPT // LANES, unroll=4)
        def _(j):
            idx = eh_v[pl.ds(ebase + j * LANES, LANES)]
            fe_v[pl.ds(j * LANES, LANES)] = plsc.load_gather(f_v, [idx])

        pltpu.sync_copy(slab_lo, mlo_hbm.at[pl.ds(hbase * half, slab_words)])
        pltpu.sync_copy(slab_hi, mhi_hbm.at[pl.ds(hbase * half, slab_words)])
        pltpu.sync_copy(fe_v, fe_hbm.at[pl.ds(ebase, EPT)])

    return body(eh_pad, ep_pad, f_pad)


def _tc_body(x_ref, f_ref, yi_ref, ys_ref, ep_ref, eh_ref, fe_ref,
             mlo_ref, mhi_ref, out_ref):
    hi = jax.lax.Precision.HIGHEST
    iota_k = lax.broadcasted_iota(jnp.int32, (K_OBJ, 1), 0)
    lane128 = lax.broadcasted_iota(jnp.int32, (1, 128), 1)
    lane1024 = lax.broadcasted_iota(jnp.int32, (1, 1024), 1)

    row10 = lax.broadcasted_iota(jnp.int32, (10, 1024), 0)
    col10 = lax.broadcasted_iota(jnp.int32, (10, 1024), 1)
    hvalid_all = (row10 * 1024 + col10) < N_HIT
    yi = yi_ref[...]
    ys = ys_ref[...]
    fh = f_ref[...]
    bkg = (yi == -1) & (ys >= 0) & hvalid_all
    n_bkg = jnp.sum(jnp.where(bkg, 1.0, 0.0))
    f_bkg = jnp.sum(jnp.where(bkg, fh, 0.0))

    def seg_chunk(c, carry):
        M, P = carry
        ep_c = ep_ref[pl.ds(c, 1), :]
        fe_c = fe_ref[pl.ds(c, 1), :]
        eh_c = eh_ref[pl.ds(c, 1), :]
        evalid = (c * 128 + lane128) < N_EDGE
        oh = (ep_c == iota_k) & evalid
        cand = jnp.where(oh, fe_c, -1.0)
        better = cand > M
        eidx = c * 128 + lane128
        pack = eidx * 16384 + eh_c
        return (jnp.where(better, cand, M), jnp.where(better, pack, P))

    M, P = lax.fori_loop(
        0, EPAD // 128, seg_chunk,
        (jnp.full((K_OBJ, 128), -1.0, jnp.float32),
         jnp.full((K_OBJ, 128), BIG_I, jnp.int32)))
    seg_max = jnp.max(M, axis=1, keepdims=True)
    p_m = jnp.where(M == seg_max, P, BIG_I)
    gp = jnp.min(p_m, axis=1, keepdims=True)
    centers = gp & 16383

    def xc_chunk(b, carry):
        hidx = b * 1024 + lane1024
        oh16 = (centers == hidx).astype(jnp.bfloat16)
        xb = x_ref[pl.ds(b * 1024, 1024), :]
        xb_hi = xb.astype(jnp.bfloat16)
        xb_lo = (xb - xb_hi.astype(jnp.float32)).astype(jnp.bfloat16)
        dn = (((1,), (0,)), ((), ()))
        acc = lax.dot_general(oh16, xb_hi, dn,
                              preferred_element_type=jnp.float32)
        acc = acc + lax.dot_general(oh16, xb_lo, dn,
                                    preferred_element_type=jnp.float32)
        return carry + acc

    xc = lax.fori_loop(0, NPAD // 1024, xc_chunk,
                       jnp.zeros((K_OBJ, DIM), jnp.float32))
    xc2 = jnp.sum(xc * xc, axis=1, keepdims=True)
    xc_hi16 = xc.astype(jnp.bfloat16)
    xc_lo16 = (xc - xc_hi16.astype(jnp.float32)).astype(jnp.bfloat16)

    f_centers = seg_max
    t = 0.5 * jnp.log((1.0 + f_centers) / (1.0 - f_centers))
    qc = t * t + 0.5

    eye_k = (iota_k == lax.broadcasted_iota(jnp.int32, (K_OBJ, K_OBJ), 1))
    eye_k = eye_k.astype(jnp.float32)
    xc2_row = lax.dot_general(xc2, eye_k, (((0,), (0,)), ((), ())),
                              precision=hi)
    qc_row = lax.dot_general(qc, eye_k, (((0,), (0,)), ((), ())),
                             precision=hi)

    qc_lo = lax.slice(qc_row, (0, 0), (1, 128))
    qc_hi = lax.slice(qc_row, (0, 128), (1, 256))

    def dense_block(b, vacc):
        xb = x_ref[pl.ds(b * 1024, 1024), :]
        x2b = jnp.sum(xb * xb, axis=1, keepdims=True)
        xb_hi = xb.astype(jnp.bfloat16)
        xb_lo = (xb - xb_hi.astype(jnp.float32)).astype(jnp.bfloat16)
        dn = (((1,), (1,)), ((), ()))
        dots = (lax.dot_general(xb_hi, xc_hi16, dn,
                                preferred_element_type=jnp.float32)
                + lax.dot_general(xb_hi, xc_lo16, dn,
                                  preferred_element_type=jnp.float32)
                + lax.dot_general(xb_lo, xc_hi16, dn,
                                  preferred_element_type=jnp.float32))
        dist = x2b + xc2_row - 2.0 * dots
        d_lo = lax.slice(dist, (0, 0), (1024, 128))
        d_hi = lax.slice(dist, (0, 128), (1024, 256))
        mlo = mlo_ref[pl.ds(b * 1024, 1024), :]
        mhi = mhi_ref[pl.ds(b * 1024, 1024), :]
        v_lo = jnp.where(mlo > 0.5, d_lo, jnp.maximum(1.0 - d_lo, 0.0))
        v_hi = jnp.where(mhi > 0.5, d_hi, jnp.maximum(1.0 - d_hi, 0.0))
        wsum = (jnp.sum(v_lo * qc_lo, axis=1, keepdims=True)
                + jnp.sum(v_hi * qc_hi, axis=1, keepdims=True))
        f_b = f_ref[pl.ds(b, 1), :]
        hvalid = (b * 1024 + lane1024) < N_HIT
        tq = 0.5 * jnp.log((1.0 + f_b) / (1.0 - f_b))
        q_b = jnp.where(hvalid, tq * tq + 0.5, 0.0)
        q_hi = q_b.astype(jnp.bfloat16)
        q_lo = (q_b - q_hi.astype(jnp.float32)).astype(jnp.bfloat16)
        w_hi = wsum.astype(jnp.bfloat16)
        w_lo = (wsum - w_hi.astype(jnp.float32)).astype(jnp.bfloat16)
        dnc = (((1,), (0,)), ((), ()))
        contrib = (lax.dot_general(q_hi, w_hi, dnc,
                                   preferred_element_type=jnp.float32)
                   + lax.dot_general(q_hi, w_lo, dnc,
                                     preferred_element_type=jnp.float32)
                   + lax.dot_general(q_lo, w_hi, dnc,
                                     preferred_element_type=jnp.float32))
        return vacc + contrib

    vtot = lax.fori_loop(0, NPAD // 1024, dense_block,
                         jnp.zeros((1, 1), jnp.float32))
    v = vtot[0, 0] / N_HIT

    b_out = (1.0 - jnp.sum(f_centers) / K_OBJ
             + jnp.where(n_bkg > 0.0, f_bkg / jnp.maximum(n_bkg, 1.0), 0.0))

    row8 = lax.broadcasted_iota(jnp.int32, (8, 128), 0)
    col8 = lax.broadcasted_iota(jnp.int32, (8, 128), 1)
    out = jnp.where((row8 == 0) & (col8 == 0), b_out,
                    jnp.where((row8 == 0) & (col8 == 1), v, 0.0))
    out_ref[...] = out


def _tc_call(x, f10, yi10, ys10, ep80, eh80, fe80, mlo, mhi):
    return pl.pallas_call(
        _tc_body,
        out_shape=jax.ShapeDtypeStruct((8, 128), jnp.float32),
    )(x, f10, yi10, ys10, ep80, eh80, fe80, mlo, mhi)


def kernel(x, f, y_i, y_s, n_true, e_true):
    eh = e_true[0]
    ep = e_true[1]
    pad_e = EPAD - N_EDGE
    pad_h = NPAD - N_HIT
    eh_pad = jnp.pad(eh, (0, pad_e))
    ep_pad = jnp.pad(ep, (0, pad_e))
    f_pad = jnp.pad(f, (0, pad_h))

    mlo_flat, mhi_flat, fe = _sc_mask_and_gather(eh_pad, ep_pad, f_pad)
    mlo = mlo_flat.reshape(NPAD, 128)
    mhi = mhi_flat.reshape(NPAD, 128)

    xp = jnp.pad(x, ((0, pad_h), (0, 0)))
    f10 = f_pad.reshape(10, 1024)
    yi10 = jnp.pad(y_i, (0, pad_h)).reshape(10, 1024)
    ys10 = jnp.pad(y_s, (0, pad_h)).reshape(10, 1024)
    ep80 = ep_pad.reshape(80, 128)
    eh80 = eh_pad.reshape(80, 128)
    fe80 = fe.reshape(80, 128)

    res = _tc_call(xp, f10, yi10, ys10, ep80, eh80, fe80, mlo, mhi)
    return res[0, 0:2]
